# baseline (speedup 1.0000x reference)
import functools
import math

import jax
import jax.numpy as jnp
from jax import lax
from jax.experimental import pallas as pl
from jax.experimental.pallas import tpu as pltpu
from jax.experimental.pallas import tpu_sc as plsc


@functools.lru_cache(maxsize=None)
def _make_gather(V, D, B):
    info = plsc.get_sparse_core_info()
    NC, NS, L = info.num_cores, info.num_subcores, info.num_lanes
    NW = NC * NS
    assert D % L == 0 and B % (8 * NW) == 0
    b_per_w = B // NW
    C = 16
    n_chunks = b_per_w // C
    scale = math.sqrt(D)
    mesh = plsc.VectorSubcoreMesh(core_axis_name="c", subcore_axis_name="s")

    nbuf = 2

    @functools.partial(
        pl.kernel,
        mesh=mesh,
        out_type=jax.ShapeDtypeStruct((B, D), jnp.float32),
        scratch_types=[
            pltpu.VMEM((b_per_w,), jnp.int32),
        ]
        + [pltpu.VMEM((C, D), jnp.float32) for _ in range(2 * nbuf)]
        + [pltpu.SemaphoreType.DMA for _ in range(2 * nbuf)],
    )
    def gather_scale(table_hbm, idx_hbm, out_hbm, idx_v, *bufs_and_sems):
        abuf = bufs_and_sems[:nbuf]
        bbuf = bufs_and_sems[nbuf : 2 * nbuf]
        gsem = bufs_and_sems[2 * nbuf : 3 * nbuf]
        ssem = bufs_and_sems[3 * nbuf : 4 * nbuf]
        wid = lax.axis_index("s") * NC + lax.axis_index("c")
        base = wid * b_per_w
        pltpu.sync_copy(idx_hbm.at[pl.ds(base, b_per_w)], idx_v)

        def scale_buf(src, dst):
            @plsc.parallel_loop(0, C, 1, unroll=2)
            def _(i):
                for j in range(D // L):
                    dst[i, pl.ds(j * L, L)] = src[i, pl.ds(j * L, L)] * scale

        def start_gather(c, b):
            pltpu.async_copy(
                table_hbm.at[idx_v.at[pl.ds(c * C, C)]], abuf[b], gsem[b]
            )

        def phase(c, b, first, last):
            pltpu.make_async_copy(
                table_hbm.at[idx_v.at[pl.ds(c * C, C)]], abuf[b], gsem[b]
            ).wait()
            if not first:
                pltpu.make_async_copy(
                    bbuf[b], out_hbm.at[pl.ds(base, C)], ssem[b]
                ).wait()
            scale_buf(abuf[b], bbuf[b])
            pltpu.async_copy(
                bbuf[b], out_hbm.at[pl.ds(base + c * C, C)], ssem[b]
            )
            if not last:
                start_gather(c + nbuf, b)

        n_groups = n_chunks // nbuf
        for b in range(nbuf):
            start_gather(b, b)
        for b in range(nbuf):
            phase(b, b, first=True, last=False)

        def group_body(g, carry):
            for b in range(nbuf):
                phase(g * nbuf + b, b, first=False, last=False)
            return carry

        lax.fori_loop(1, n_groups - 1, group_body, 0)
        for b in range(nbuf):
            phase((n_groups - 1) * nbuf + b, b, first=False, last=True)
        for b in range(nbuf):
            pltpu.make_async_copy(
                bbuf[b], out_hbm.at[pl.ds(base, C)], ssem[b]
            ).wait()

    return gather_scale


def kernel(x, emb):
    V, D = emb.shape
    B = x.size
    x_flat = x.reshape(B).astype(jnp.int32)
    out = _make_gather(V, D, B)(emb, x_flat)
    return out.reshape(x.shape + (D,))


# speedup vs baseline: 1.2524x; 1.2524x over previous
import functools
import math

import jax
import jax.numpy as jnp
from jax import lax
from jax.experimental import pallas as pl
from jax.experimental.pallas import tpu as pltpu
from jax.experimental.pallas import tpu_sc as plsc


@functools.lru_cache(maxsize=None)
def _make_gather(V, D, B):
    info = plsc.get_sparse_core_info()
    NC, NS, L = info.num_cores, info.num_subcores, info.num_lanes
    NW = NC * NS
    assert D % L == 0 and B % (8 * NW) == 0
    b_per_w = B // NW
    C = 32
    H = C // 2
    n_chunks = b_per_w // C
    scale = math.sqrt(D)
    mesh = plsc.VectorSubcoreMesh(core_axis_name="c", subcore_axis_name="s")

    @functools.partial(
        pl.kernel,
        mesh=mesh,
        out_type=jax.ShapeDtypeStruct((B, D), jnp.float32),
        scratch_types=[
            pltpu.VMEM((b_per_w,), jnp.int32),
            pltpu.VMEM((C, D), jnp.float32),
            pltpu.VMEM((C, D), jnp.float32),
            pltpu.VMEM((H, D), jnp.float32),
            pltpu.VMEM((H, D), jnp.float32),
            pltpu.SemaphoreType.DMA,
            pltpu.SemaphoreType.DMA,
            pltpu.SemaphoreType.DMA,
            pltpu.SemaphoreType.DMA,
        ],
    )
    def gather_scale(table_hbm, idx_hbm, out_hbm, idx_v,
                     a0, a1, b0, b1, g0, g1, s0, s1):
        abuf, bbuf = (a0, a1), (b0, b1)
        gsem, ssem = (g0, g1), (s0, s1)
        wid = lax.axis_index("s") * NC + lax.axis_index("c")
        base = wid * b_per_w
        pltpu.sync_copy(idx_hbm.at[pl.ds(base, b_per_w)], idx_v)

        def scale_half(src, h, dst):
            @plsc.parallel_loop(0, H, 1)
            def _(i):
                for j in range(D // L):
                    dst[i, pl.ds(j * L, L)] = (
                        src[h * H + i, pl.ds(j * L, L)] * scale
                    )

        def start_gather(c):
            return pltpu.async_copy(
                table_hbm.at[idx_v.at[pl.ds(c * C, C)]],
                abuf[c % 2],
                gsem[c % 2],
            )

        gather = [None] * n_chunks
        store = {}
        gather[0] = start_gather(0)
        gather[1] = start_gather(1)
        for c in range(n_chunks):
            p = c % 2
            gather[c].wait()
            for h in range(2):
                if c > 0:
                    store[(c - 1, h)].wait()
                scale_half(abuf[p], h, bbuf[h])
                store[(c, h)] = pltpu.async_copy(
                    bbuf[h],
                    out_hbm.at[pl.ds(base + c * C + h * H, H)],
                    ssem[h],
                )
            if c + 2 < n_chunks:
                gather[c + 2] = start_gather(c + 2)
        store[(n_chunks - 1, 0)].wait()
        store[(n_chunks - 1, 1)].wait()

    return gather_scale


def kernel(x, emb):
    V, D = emb.shape
    B = x.size
    x_flat = x.reshape(B).astype(jnp.int32)
    out = _make_gather(V, D, B)(emb, x_flat)
    return out.reshape(x.shape + (D,))


# speedup vs baseline: 1.5324x; 1.2236x over previous
import functools
import math

import jax
import jax.numpy as jnp
from jax import lax
from jax.experimental import pallas as pl
from jax.experimental.pallas import tpu as pltpu
from jax.experimental.pallas import tpu_sc as plsc


@functools.lru_cache(maxsize=None)
def _make_gather(V, D, B):
    info = plsc.get_sparse_core_info()
    NC, NS, L = info.num_cores, info.num_subcores, info.num_lanes
    NW = NC * NS
    assert D % L == 0 and B % (8 * NW) == 0
    b_per_w = B // NW
    C = 16
    n_chunks = b_per_w // C
    n_groups = n_chunks // 2
    scale = math.sqrt(D)
    mesh = plsc.VectorSubcoreMesh(core_axis_name="c", subcore_axis_name="s")

    @functools.partial(
        pl.kernel,
        mesh=mesh,
        out_type=jax.ShapeDtypeStruct((B, D), jnp.float32),
        scratch_types=[
            pltpu.VMEM((b_per_w,), jnp.int32),
            pltpu.VMEM((C, D), jnp.float32),
            pltpu.VMEM((C, D), jnp.float32),
            pltpu.VMEM((C, D), jnp.float32),
            pltpu.VMEM((C, D), jnp.float32),
            pltpu.SemaphoreType.DMA,
            pltpu.SemaphoreType.DMA,
            pltpu.SemaphoreType.DMA,
            pltpu.SemaphoreType.DMA,
        ],
    )
    def gather_scale(table_hbm, idx_hbm, out_hbm, idx_v,
                     a0, a1, b0, b1, g0, g1, s0, s1):
        abuf, bbuf = (a0, a1), (b0, b1)
        gsem, ssem = (g0, g1), (s0, s1)
        wid = lax.axis_index("s") * NC + lax.axis_index("c")
        base = wid * b_per_w
        pltpu.sync_copy(idx_hbm.at[pl.ds(base, b_per_w)], idx_v)

        def gather_descr(c, b):
            return pltpu.make_async_copy(
                table_hbm.at[idx_v.at[pl.ds(c * C, C)]], abuf[b], gsem[b]
            )

        def store_descr(c, b):
            return pltpu.make_async_copy(
                bbuf[b], out_hbm.at[pl.ds(base + c * C, C)], ssem[b]
            )

        def phase(c, b):
            gather_descr(c, b).wait()

            @pl.when(c >= 2)
            def _():
                store_descr(c, b).wait()

            @plsc.parallel_loop(0, C, 1)
            def _(i):
                for j in range(D // L):
                    bbuf[b][i, pl.ds(j * L, L)] = (
                        abuf[b][i, pl.ds(j * L, L)] * scale
                    )

            store_descr(c, b).start()

            @pl.when(c + 2 < n_chunks)
            def _():
                gather_descr(c + 2, b).start()

        gather_descr(0, 0).start()
        gather_descr(1, 1).start()

        def group_body(g, carry):
            phase(g * 2, 0)
            phase(g * 2 + 1, 1)
            return carry

        lax.fori_loop(0, n_groups, group_body, 0)
        store_descr(0, 0).wait()
        store_descr(0, 1).wait()

    return gather_scale


def kernel(x, emb):
    V, D = emb.shape
    B = x.size
    x_flat = x.reshape(B).astype(jnp.int32)
    out = _make_gather(V, D, B)(emb, x_flat)
    return out.reshape(x.shape + (D,))
